# Initial kernel scaffold; baseline (speedup 1.0000x reference)
import jax
import jax.numpy as jnp
from jax.experimental import pallas as pl


def kernel(h_nodes, bias, wif, whf, bif, bhf, wir, whr, bir, bhr):
    raise NotImplementedError("write your pallas kernel here")


# speedup vs baseline: 1.4381x; 1.4381x over previous
import math

import jax
import jax.numpy as jnp
import numpy as np
from jax import lax
from jax.experimental import pallas as pl
from jax.experimental.pallas import tpu as pltpu

_NUMS = np.asarray([40 + (i % 41) for i in range(256)], np.int64)
_B = int(_NUMS.shape[0])
_T = int(_NUMS.max())
_N = int(_NUMS.sum())
_H = 300
_HP = 384
_BP = _B

_STARTS = np.concatenate([[0], np.cumsum(_NUMS)[:-1]])
_GRAPH_ID = np.repeat(np.arange(_B), _NUMS)
_WITHIN = np.arange(_N) - _STARTS[_GRAPH_ID]
_POS_NP = (_WITHIN * _BP + _GRAPH_ID).astype(np.int32)
_INV_NP = np.zeros(_T * _BP, np.int32)
_INV_NP[_POS_NP] = np.arange(_N, dtype=np.int32)
_ROWVALID_NP = np.zeros(_T * _BP, bool)
_ROWVALID_NP[_POS_NP] = True

_POS = jnp.asarray(_POS_NP)
_INV = jnp.asarray(_INV_NP)
_ROWVALID = jnp.asarray(_ROWVALID_NP[:, None])
_LENF = jnp.asarray(_NUMS.astype(np.float32)[:, None])


def _h0_kernel(hpad_ref, lenf_ref, h0_ref, acc_ref):
    t = pl.program_id(0)
    T = pl.num_programs(0)

    @pl.when(t == 0)
    def _():
        acc_ref[...] = jnp.full_like(acc_ref, -1e9)

    tf = lax.convert_element_type(t, jnp.float32)
    valid = tf < lenf_ref[...]
    x = hpad_ref[0]
    acc_ref[...] = jnp.maximum(acc_ref[...], jnp.where(valid, x, -1e9))

    @pl.when(t == T - 1)
    def _():
        h0_ref[...] = acc_ref[...]


def _bigru_kernel(hf_ref, hr_ref, lenf_ref, bias_ref, h0_ref,
                  wif_ref, bif_ref, whf_ref, bhf_ref,
                  wir_ref, bir_ref, whr_ref, bhr_ref,
                  outf_ref, outr_ref, poolf_ref, poolr_ref,
                  sf_ref, sr_ref, pf_ref, pr_ref):
    t = pl.program_id(0)
    T = pl.num_programs(0)
    Hp = h0_ref.shape[-1]

    @pl.when(t == 0)
    def _():
        h0 = h0_ref[...]
        sf_ref[...] = h0
        sr_ref[...] = h0
        pf_ref[...] = jnp.zeros_like(pf_ref)
        pr_ref[...] = jnp.zeros_like(pr_ref)

    lenb = lenf_ref[...]
    tf = lax.convert_element_type(t, jnp.float32)
    trf = lax.convert_element_type(T - 1 - t, jnp.float32)
    valid_f = (tf < lenb).astype(jnp.float32)
    valid_r = (trf < lenb).astype(jnp.float32)
    bias3 = bias_ref[...]

    def cell(x_ref, valid, h, wi_ref, bi_ref, wh_ref, bh_ref):
        msg = jnp.maximum(x_ref[0] + bias3, 0.0) * valid
        gi = jnp.dot(msg, wi_ref[...], preferred_element_type=jnp.float32) + bi_ref[...]
        gh = jnp.dot(h, wh_ref[...], preferred_element_type=jnp.float32) + bh_ref[...]
        r = jax.nn.sigmoid(gi[:, 0:Hp] + gh[:, 0:Hp])
        z = jax.nn.sigmoid(gi[:, Hp:2 * Hp] + gh[:, Hp:2 * Hp])
        n = jnp.tanh(gi[:, 2 * Hp:] + r * gh[:, 2 * Hp:])
        return (1.0 - z) * n + z * h

    h_f = cell(hf_ref, valid_f, sf_ref[...], wif_ref, bif_ref, whf_ref, bhf_ref)
    h_r = cell(hr_ref, valid_r, sr_ref[...], wir_ref, bir_ref, whr_ref, bhr_ref)

    sf_ref[...] = h_f
    sr_ref[...] = h_r
    outf_ref[0] = h_f
    outr_ref[0] = h_r
    pf_ref[...] = pf_ref[...] + h_f * valid_f
    pr_ref[...] = pr_ref[...] + h_r * valid_r

    @pl.when(t == T - 1)
    def _():
        inv = pl.reciprocal(jnp.maximum(lenb, 1.0), approx=True)
        poolf_ref[...] = pf_ref[...] * inv
        poolr_ref[...] = pr_ref[...] * inv


def _pad_w(w, H, Hp):
    pad = Hp - H
    wt = w.T
    gates = [jnp.pad(wt[:, g * H:(g + 1) * H], ((0, pad), (0, pad)))
             for g in range(3)]
    return jnp.concatenate(gates, axis=1)


def _pad_b(b, H, Hp):
    pad = Hp - H
    gates = [jnp.pad(b[g * H:(g + 1) * H], (0, pad)) for g in range(3)]
    return jnp.concatenate(gates, axis=0)[None, :]


def _run(hpad, lenf, bias_p, wif_p, bif_p, whf_p, bhf_p, wir_p, bir_p,
         whr_p, bhr_p):
    T, B, Hp = hpad.shape
    H3 = 3 * Hp

    h0_v = pl.pallas_call(
        _h0_kernel,
        grid=(T,),
        in_specs=[
            pl.BlockSpec((1, B, Hp), lambda t: (t, 0, 0)),
            pl.BlockSpec((B, 1), lambda t: (0, 0)),
        ],
        out_specs=pl.BlockSpec((B, Hp), lambda t: (0, 0)),
        out_shape=jax.ShapeDtypeStruct((B, Hp), jnp.float32),
        scratch_shapes=[pltpu.VMEM((B, Hp), jnp.float32)],
        compiler_params=pltpu.CompilerParams(
            dimension_semantics=("arbitrary",)),
    )(hpad, lenf)

    fixed = lambda t: (0, 0)
    return pl.pallas_call(
        _bigru_kernel,
        grid=(T,),
        in_specs=[
            pl.BlockSpec((1, B, Hp), lambda t: (t, 0, 0)),
            pl.BlockSpec((1, B, Hp), lambda t: (T - 1 - t, 0, 0)),
            pl.BlockSpec((B, 1), fixed),
            pl.BlockSpec((1, Hp), fixed),
            pl.BlockSpec((B, Hp), fixed),
            pl.BlockSpec((Hp, H3), fixed),
            pl.BlockSpec((1, H3), fixed),
            pl.BlockSpec((Hp, H3), fixed),
            pl.BlockSpec((1, H3), fixed),
            pl.BlockSpec((Hp, H3), fixed),
            pl.BlockSpec((1, H3), fixed),
            pl.BlockSpec((Hp, H3), fixed),
            pl.BlockSpec((1, H3), fixed),
        ],
        out_specs=(
            pl.BlockSpec((1, B, Hp), lambda t: (t, 0, 0)),
            pl.BlockSpec((1, B, Hp), lambda t: (T - 1 - t, 0, 0)),
            pl.BlockSpec((B, Hp), fixed),
            pl.BlockSpec((B, Hp), fixed),
        ),
        out_shape=(
            jax.ShapeDtypeStruct((T, B, Hp), jnp.float32),
            jax.ShapeDtypeStruct((T, B, Hp), jnp.float32),
            jax.ShapeDtypeStruct((B, Hp), jnp.float32),
            jax.ShapeDtypeStruct((B, Hp), jnp.float32),
        ),
        scratch_shapes=[
            pltpu.VMEM((B, Hp), jnp.float32),
            pltpu.VMEM((B, Hp), jnp.float32),
            pltpu.VMEM((B, Hp), jnp.float32),
            pltpu.VMEM((B, Hp), jnp.float32),
        ],
        compiler_params=pltpu.CompilerParams(
            dimension_semantics=("arbitrary",)),
    )(hpad, hpad, lenf, bias_p, h0_v,
      wif_p, bif_p, whf_p, bhf_p, wir_p, bir_p, whr_p, bhr_p)


def kernel(h_nodes, bias, wif, whf, bif, bhf, wir, whr, bir, bhr):
    H, Hp, T, B = _H, _HP, _T, _BP

    h_p = jnp.pad(h_nodes.astype(jnp.float32), ((0, 0), (0, Hp - H)))
    hpad = jnp.where(_ROWVALID, h_p[_INV], 0.0).reshape(T, B, Hp)

    bias_p = jnp.pad(bias, (0, Hp - H))[None, :]
    out_f, out_r, pool_f, pool_r = _run(
        hpad, _LENF, bias_p,
        _pad_w(wif, H, Hp), _pad_b(bif, H, Hp),
        _pad_w(whf, H, Hp), _pad_b(bhf, H, Hp),
        _pad_w(wir, H, Hp), _pad_b(bir, H, Hp),
        _pad_w(whr, H, Hp), _pad_b(bhr, H, Hp))

    rows_f = out_f.reshape(T * B, Hp)
    rows_r = out_r.reshape(T * B, Hp)
    node_out = jnp.concatenate(
        [jnp.take(rows_f, _POS, axis=0)[:, :H],
         jnp.take(rows_r, _POS, axis=0)[:, :H]], axis=-1)
    pooled = jnp.concatenate([pool_f[:, :H], pool_r[:, :H]], axis=-1)
    return node_out, pooled
